# Initial kernel scaffold; baseline (speedup 1.0000x reference)
import jax
import jax.numpy as jnp
from jax.experimental import pallas as pl


def kernel(x, edge_index, edge_attr, batch_idx, We, W1, b1, W2, b2):
    raise NotImplementedError("write your pallas kernel here")


# speedup vs baseline: 2.3888x; 2.3888x over previous
import functools

import jax
import jax.numpy as jnp
from jax import lax
from jax.experimental import pallas as pl
from jax.experimental.pallas import tpu as pltpu
from jax.experimental.pallas import tpu_sc as plsc


def _edge_matmul(edge_attr, We):
    E, DE = edge_attr.shape
    D = We.shape[1]
    BE = 3200

    def body(ea_ref, we_ref, z_ref):
        z_ref[...] = jnp.dot(ea_ref[...], we_ref[...],
                             preferred_element_type=jnp.float32)

    return pl.pallas_call(
        body,
        grid=(E // BE,),
        in_specs=[pl.BlockSpec((BE, DE), lambda i: (i, 0)),
                  pl.BlockSpec((DE, D), lambda i: (0, 0))],
        out_specs=pl.BlockSpec((BE, D), lambda i: (i, 0)),
        out_shape=jax.ShapeDtypeStruct((E, D), jnp.float32),
    )(edge_attr, We)


def _sc_aggregate(x, src, dst, z):
    N, D = x.shape
    E = src.shape[0]
    info = plsc.get_sparse_core_info()
    NC, NS = info.num_cores, info.num_subcores
    NW = NC * NS
    EW = E // NW
    C = 80
    NCH = EW // C
    RB = 128
    NP = ((N + RB * NS - 1) // (RB * NS)) * (RB * NS)
    RN = NP // NS
    NRB = RN // RB
    mesh = plsc.VectorSubcoreMesh(core_axis_name="c", subcore_axis_name="s")

    @functools.partial(
        pl.kernel,
        out_type=jax.ShapeDtypeStruct((NC, NP, D), jnp.float32),
        mesh=mesh,
        scratch_types=[
            pltpu.VMEM((C,), jnp.int32),
            pltpu.VMEM((C,), jnp.int32),
            pltpu.VMEM((C, D), jnp.float32),
            pltpu.VMEM((C, D), jnp.float32),
            pltpu.VMEM((RB, D), jnp.float32),
            pltpu.VMEM_SHARED((NP, D), jnp.float32),
            pltpu.SemaphoreType.DMA,
        ],
    )
    def agg_kernel(x_hbm, src_hbm, dst_hbm, z_hbm, out_hbm,
                   sidx, didx, rows, zrows, obuf, agg_sh, sem):
        cid = lax.axis_index("c")
        sid = lax.axis_index("s")
        wid = sid * NC + cid
        r0 = sid * RN

        def zero_row(i, carry):
            for k in range(D // 16):
                obuf[i, pl.ds(k * 16, 16)] = jnp.zeros((16,), jnp.float32)
            return carry
        lax.fori_loop(0, RB, zero_row, 0)
        for j in range(NRB):
            pltpu.sync_copy(obuf, agg_sh.at[pl.ds(r0 + j * RB, RB)])
        plsc.subcore_barrier()

        base = wid * EW

        def chunk_body(ci, carry):
            e0 = base + ci * C
            pltpu.sync_copy(src_hbm.at[pl.ds(e0, C)], sidx)
            pltpu.sync_copy(dst_hbm.at[pl.ds(e0, C)], didx)
            pltpu.async_copy(x_hbm.at[sidx], rows, sem).wait()
            pltpu.sync_copy(z_hbm.at[pl.ds(e0, C)], zrows)

            def row_body(i, c2):
                for k in range(D // 16):
                    sl = pl.ds(k * 16, 16)
                    rows[i, sl] = jnp.maximum(rows[i, sl] + zrows[i, sl], 0.0)
                return c2
            lax.fori_loop(0, C, row_body, 0)
            pltpu.sync_copy(rows, agg_sh.at[didx], add=True)
            return carry
        lax.fori_loop(0, NCH, chunk_body, 0)

        plsc.subcore_barrier()
        for j in range(NRB):
            pltpu.sync_copy(agg_sh.at[pl.ds(r0 + j * RB, RB)], obuf)
            pltpu.sync_copy(obuf, out_hbm.at[cid, pl.ds(r0 + j * RB, RB)])

    parts = agg_kernel(x, src, dst, z)
    return parts[:, :N]


def _mlp(x, a0, a1, W1, b1, W2, b2):
    N, D = x.shape
    BN = 2000

    def body(x_ref, a0_ref, a1_ref, w1_ref, b1_ref, w2_ref, b2_ref, o_ref):
        h = x_ref[...] + a0_ref[...] + a1_ref[...]
        h = jnp.maximum(
            jnp.dot(h, w1_ref[...], preferred_element_type=jnp.float32)
            + b1_ref[...], 0.0)
        o_ref[...] = jnp.dot(h, w2_ref[...],
                             preferred_element_type=jnp.float32) + b2_ref[...]

    return pl.pallas_call(
        body,
        grid=(N // BN,),
        in_specs=[pl.BlockSpec((BN, D), lambda i: (i, 0)),
                  pl.BlockSpec((BN, D), lambda i: (i, 0)),
                  pl.BlockSpec((BN, D), lambda i: (i, 0)),
                  pl.BlockSpec((D, D), lambda i: (0, 0)),
                  pl.BlockSpec((1, D), lambda i: (0, 0)),
                  pl.BlockSpec((D, D), lambda i: (0, 0)),
                  pl.BlockSpec((1, D), lambda i: (0, 0))],
        out_specs=pl.BlockSpec((BN, D), lambda i: (i, 0)),
        out_shape=jax.ShapeDtypeStruct((N, D), jnp.float32),
    )(x, a0, a1, W1, b1.reshape(1, D), W2, b2.reshape(1, D))


def kernel(x, edge_index, edge_attr, batch_idx, We, W1, b1, W2, b2):
    src = edge_index[0]
    dst = edge_index[1]
    z = _edge_matmul(edge_attr, We)
    parts = _sc_aggregate(x, src, dst, z)
    return _mlp(x, parts[0], parts[1], W1, b1, W2, b2)
